# baseline (speedup 1.0000x reference)
import functools

import jax
import jax.numpy as jnp
from jax.experimental import pallas as pl
from jax.experimental.pallas import tpu as pltpu

_DIM = 64
_L = 8
_BETA = 0.25
_EPS = 1e-10


def _ce(a, b):
    return jnp.minimum(a, b), jnp.maximum(a, b)


def _merge22(a, b):
    c0, t = _ce(a[0], b[0])
    s, c3 = _ce(a[1], b[1])
    c1, c2 = _ce(t, s)
    return (c0, c1, c2, c3)


def _merge44(a, b):
    e = _merge22((a[0], a[2]), (b[0], b[2]))
    o = _merge22((a[1], a[3]), (b[1], b[3]))
    c1, c2 = _ce(o[0], e[1])
    c3, c4 = _ce(o[1], e[2])
    c5, c6 = _ce(o[2], e[3])
    return (e[0], c1, c2, c3, c4, c5, c6, o[3])


def _merge88_low(a, b):
    c = [jnp.minimum(a[i], b[7 - i]) for i in range(8)]
    for d in (4, 2, 1):
        nc = list(c)
        for i in range(8):
            if (i // d) % 2 == 0:
                nc[i], nc[i + d] = _ce(c[i], c[i + d])
        c = nc
    return tuple(c)


def _kth_smallest(work, l):
    r, kk = work.shape
    n_cols = kk // 128
    assert n_cols % 8 == 0 and l == 8
    cols = [work[:, i * 128:(i + 1) * 128] for i in range(n_cols)]
    s2 = [_ce(cols[2 * i], cols[2 * i + 1]) for i in range(n_cols // 2)]
    s4 = [_merge22(s2[2 * i], s2[2 * i + 1]) for i in range(len(s2) // 2)]
    s8 = [_merge44(s4[2 * i], s4[2 * i + 1]) for i in range(len(s4) // 2)]
    while len(s8) > 1:
        s8 = [_merge88_low(s8[2 * i], s8[2 * i + 1]) for i in range(len(s8) // 2)]
    cand = jnp.concatenate(s8[0], axis=1)
    pos_inf = jnp.float32(jnp.inf)
    for _ in range(l - 1):
        m = jnp.min(cand, axis=1, keepdims=True)
        cand = jnp.where(cand == m, pos_inf, cand)
    return jnp.min(cand, axis=1, keepdims=True)


def _fused_body(flat_ref, cat_wt_ref, lin_b_ref,
                rep_ref, z_ref, loss_ref, perp_ref,
                counts_ref, sumsq_ref, *, n_total, n_blocks):
    i = pl.program_id(0)
    x = flat_ref[...]
    cat_wt = cat_wt_ref[...]
    r, _ = x.shape
    k = cat_wt.shape[1] // 2
    dict_wt = cat_wt[:, k:]

    big = jax.lax.dot_general(
        x, cat_wt, (((1,), (0,)), ((), ())),
        preferred_element_type=jnp.float32)
    logits = big[:, :k] + lin_b_ref[...]
    dots = big[:, k:]
    rowmax = jnp.max(logits, axis=1, keepdims=True)
    e = jnp.exp(logits - rowmax)
    denom = jnp.sum(e, axis=1, keepdims=True)

    dsq = jnp.sum(dict_wt * dict_wt, axis=0, keepdims=True)
    fsq = jnp.sum(x * x, axis=1, keepdims=True)
    work = fsq + dsq - 2.0 * dots

    t8 = _kth_smallest(work, _L)
    rep_ref[...] = jnp.where(work <= t8, jnp.float32(1.0), jnp.float32(0.0))

    @pl.when(jnp.sum(rep_ref[...]) > jnp.float32(r * _L) + 0.5)
    def _tie_fixup():
        iota = jax.lax.broadcasted_iota(jnp.int32, (r, k), 1)
        pos_inf = jnp.float32(jnp.inf)
        wk = work
        acc = jnp.zeros((r, k), jnp.float32)
        for _ in range(_L):
            m = jnp.min(wk, axis=1, keepdims=True)
            cand = jnp.where(wk == m, iota, k)
            first = jnp.min(cand, axis=1, keepdims=True)
            sel = iota == first
            acc += jnp.where(sel, jnp.float32(1.0), jnp.float32(0.0))
            wk = jnp.where(sel, pos_inf, wk)
        rep_ref[...] = acc

    maskf = rep_ref[...]

    scale = 1.0 / (jnp.float32(_L) * denom)
    rep = maskf * (e * scale)
    rep_ref[...] = rep

    z_dl = jax.lax.dot_general(
        rep, dict_wt, (((1,), (1,)), ((), ())),
        preferred_element_type=jnp.float32)
    diff = z_dl - x
    z_ref[...] = x + diff

    @pl.when(i == 0)
    def _init():
        counts_ref[...] = jnp.zeros_like(counts_ref)
        sumsq_ref[0, 0] = jnp.float32(0.0)

    counts_ref[...] += jnp.sum(maskf, axis=0, keepdims=True)
    sumsq_ref[0, 0] += jnp.sum(diff * diff)

    @pl.when(i == n_blocks - 1)
    def _finalize():
        loss_ref[...] = jnp.full(
            (1, 1), (_BETA / (n_total * _DIM)) * sumsq_ref[0, 0], jnp.float32)
        avg = counts_ref[...] * (1.0 / n_total)
        p = avg / jnp.sum(avg)
        perp_ref[...] = jnp.full(
            (1, 1), jnp.exp(-jnp.sum(p * jnp.log(p + _EPS))), jnp.float32)


def kernel(z_e, dict_w, lin_w, lin_b):
    b, c, h, w = z_e.shape
    n = b * h * w
    k = dict_w.shape[0]
    flat = jnp.transpose(z_e, (0, 2, 3, 1)).reshape(n, c)
    cat_wt = jnp.concatenate([lin_w.T, dict_w.T], axis=1)
    lin_b2 = lin_b.reshape(1, k)

    r = 256 if n % 256 == 0 else n
    n_blocks = n // r

    rep_sparse, z_dl, loss, perp = pl.pallas_call(
        functools.partial(_fused_body, n_total=n, n_blocks=n_blocks),
        grid=(n_blocks,),
        in_specs=[
            pl.BlockSpec((r, c), lambda i: (i, 0)),
            pl.BlockSpec((c, 2 * k), lambda i: (0, 0)),
            pl.BlockSpec((1, k), lambda i: (0, 0)),
        ],
        out_specs=[
            pl.BlockSpec((r, k), lambda i: (i, 0)),
            pl.BlockSpec((r, c), lambda i: (i, 0)),
            pl.BlockSpec((1, 1), lambda i: (0, 0)),
            pl.BlockSpec((1, 1), lambda i: (0, 0)),
        ],
        out_shape=[
            jax.ShapeDtypeStruct((n, k), jnp.float32),
            jax.ShapeDtypeStruct((n, c), jnp.float32),
            jax.ShapeDtypeStruct((1, 1), jnp.float32),
            jax.ShapeDtypeStruct((1, 1), jnp.float32),
        ],
        scratch_shapes=[
            pltpu.VMEM((1, k), jnp.float32),
            pltpu.SMEM((1, 1), jnp.float32),
        ],
    )(flat, cat_wt, lin_b2)

    z_st = jnp.transpose(z_dl.reshape(b, h, w, c), (0, 3, 1, 2))
    return (loss.reshape(()), z_st, perp.reshape(()), rep_sparse)


# speedup vs baseline: 1.0878x; 1.0878x over previous
import functools

import jax
import jax.numpy as jnp
from jax.experimental import pallas as pl
from jax.experimental.pallas import tpu as pltpu

_DIM = 64
_L = 8
_BETA = 0.25
_EPS = 1e-10


def _ce(a, b):
    return jnp.minimum(a, b), jnp.maximum(a, b)


def _merge22(a, b):
    c0, t = _ce(a[0], b[0])
    s, c3 = _ce(a[1], b[1])
    c1, c2 = _ce(t, s)
    return (c0, c1, c2, c3)


def _merge44(a, b):
    e = _merge22((a[0], a[2]), (b[0], b[2]))
    o = _merge22((a[1], a[3]), (b[1], b[3]))
    c1, c2 = _ce(o[0], e[1])
    c3, c4 = _ce(o[1], e[2])
    c5, c6 = _ce(o[2], e[3])
    return (e[0], c1, c2, c3, c4, c5, c6, o[3])


def _merge88_low(a, b):
    c = [jnp.minimum(a[i], b[7 - i]) for i in range(8)]
    for d in (4, 2, 1):
        nc = list(c)
        for i in range(8):
            if (i // d) % 2 == 0:
                nc[i], nc[i + d] = _ce(c[i], c[i + d])
        c = nc
    return tuple(c)


def _kth_smallest(work, l):
    r, kk = work.shape
    n_cols = kk // 128
    assert n_cols % 8 == 0 and l == 8
    cols = [work[:, i * 128:(i + 1) * 128] for i in range(n_cols)]
    s2 = [_ce(cols[2 * i], cols[2 * i + 1]) for i in range(n_cols // 2)]
    s4 = [_merge22(s2[2 * i], s2[2 * i + 1]) for i in range(len(s2) // 2)]
    s8 = [_merge44(s4[2 * i], s4[2 * i + 1]) for i in range(len(s4) // 2)]
    while len(s8) > 1:
        s8 = [_merge88_low(s8[2 * i], s8[2 * i + 1]) for i in range(len(s8) // 2)]
    cand = jnp.concatenate(s8[0], axis=1)
    pos_inf = jnp.float32(jnp.inf)
    for _ in range(l - 1):
        m = jnp.min(cand, axis=1, keepdims=True)
        cand = jnp.where(cand == m, pos_inf, cand)
    return jnp.min(cand, axis=1, keepdims=True)


def _fused_body(flat_ref, cat_wt_ref, lin_b_ref,
                rep_ref, z_ref, loss_ref, perp_ref,
                counts_ref, sumsq_ref, *, n_total, n_blocks):
    i = pl.program_id(0)
    x = flat_ref[...]
    cat_wt = cat_wt_ref[...]
    r, _ = x.shape
    k = cat_wt.shape[1] // 2
    dict_wt = cat_wt[:, k:]

    big = jax.lax.dot_general(
        x, cat_wt, (((1,), (0,)), ((), ())),
        preferred_element_type=jnp.float32)
    logits = big[:, :k] + lin_b_ref[...]
    dots = big[:, k:]
    rowmax = jnp.max(logits, axis=1, keepdims=True)
    e = jnp.exp(logits - rowmax)
    denom = jnp.sum(e, axis=1, keepdims=True)

    dsq = jnp.sum(dict_wt * dict_wt, axis=0, keepdims=True)
    fsq = jnp.sum(x * x, axis=1, keepdims=True)
    work = fsq + dsq - 2.0 * dots

    t8 = _kth_smallest(work, _L)
    maskf = jnp.where(work <= t8, jnp.float32(1.0), jnp.float32(0.0))
    colsum = jnp.sum(maskf, axis=0, keepdims=True)

    scale = 1.0 / (jnp.float32(_L) * denom)
    rep_ref[...] = maskf * (e * scale)

    @pl.when(i == 0)
    def _init():
        counts_ref[...] = jnp.zeros_like(counts_ref)
        sumsq_ref[0, 0] = jnp.float32(0.0)

    counts_ref[...] += colsum

    @pl.when(jnp.sum(colsum) > jnp.float32(r * _L) + 0.5)
    def _tie_fixup():
        iota = jax.lax.broadcasted_iota(jnp.int32, (r, k), 1)
        pos_inf = jnp.float32(jnp.inf)
        wk = work
        for _ in range(_L):
            m = jnp.min(wk, axis=1, keepdims=True)
            cand = jnp.where(wk == m, iota, k)
            first = jnp.min(cand, axis=1, keepdims=True)
            wk = jnp.where(iota == first, pos_inf, wk)
        acc = jnp.where(wk == pos_inf, jnp.float32(1.0), jnp.float32(0.0))
        rep_ref[...] = acc * (e * scale)
        counts_ref[...] += jnp.sum(acc, axis=0, keepdims=True) - colsum

    rep = rep_ref[...]
    z_dl = jax.lax.dot_general(
        rep, dict_wt, (((1,), (1,)), ((), ())),
        preferred_element_type=jnp.float32)
    diff = z_dl - x
    z_ref[...] = x + diff
    sumsq_ref[0, 0] += jnp.sum(diff * diff)

    @pl.when(i == n_blocks - 1)
    def _finalize():
        loss_ref[...] = jnp.full(
            (1, 1), (_BETA / (n_total * _DIM)) * sumsq_ref[0, 0], jnp.float32)
        avg = counts_ref[...] * (1.0 / n_total)
        p = avg / jnp.sum(avg)
        perp_ref[...] = jnp.full(
            (1, 1), jnp.exp(-jnp.sum(p * jnp.log(p + _EPS))), jnp.float32)


def kernel(z_e, dict_w, lin_w, lin_b):
    b, c, h, w = z_e.shape
    n = b * h * w
    k = dict_w.shape[0]
    flat = jnp.transpose(z_e, (0, 2, 3, 1)).reshape(n, c)
    cat_wt = jnp.concatenate([lin_w.T, dict_w.T], axis=1)
    lin_b2 = lin_b.reshape(1, k)

    r = 256 if n % 256 == 0 else n
    n_blocks = n // r

    rep_sparse, z_dl, loss, perp = pl.pallas_call(
        functools.partial(_fused_body, n_total=n, n_blocks=n_blocks),
        grid=(n_blocks,),
        in_specs=[
            pl.BlockSpec((r, c), lambda i: (i, 0)),
            pl.BlockSpec((c, 2 * k), lambda i: (0, 0)),
            pl.BlockSpec((1, k), lambda i: (0, 0)),
        ],
        out_specs=[
            pl.BlockSpec((r, k), lambda i: (i, 0)),
            pl.BlockSpec((r, c), lambda i: (i, 0)),
            pl.BlockSpec((1, 1), lambda i: (0, 0)),
            pl.BlockSpec((1, 1), lambda i: (0, 0)),
        ],
        out_shape=[
            jax.ShapeDtypeStruct((n, k), jnp.float32),
            jax.ShapeDtypeStruct((n, c), jnp.float32),
            jax.ShapeDtypeStruct((1, 1), jnp.float32),
            jax.ShapeDtypeStruct((1, 1), jnp.float32),
        ],
        scratch_shapes=[
            pltpu.VMEM((1, k), jnp.float32),
            pltpu.SMEM((1, 1), jnp.float32),
        ],
    )(flat, cat_wt, lin_b2)

    z_st = jnp.transpose(z_dl.reshape(b, h, w, c), (0, 3, 1, 2))
    return (loss.reshape(()), z_st, perp.reshape(()), rep_sparse)


# speedup vs baseline: 1.0991x; 1.0103x over previous
import functools

import jax
import jax.numpy as jnp
from jax.experimental import pallas as pl
from jax.experimental.pallas import tpu as pltpu

_DIM = 64
_L = 8
_BETA = 0.25
_EPS = 1e-10


def _ce(a, b):
    return jnp.minimum(a, b), jnp.maximum(a, b)


def _merge22(a, b):
    c0, t = _ce(a[0], b[0])
    s, c3 = _ce(a[1], b[1])
    c1, c2 = _ce(t, s)
    return (c0, c1, c2, c3)


def _merge44(a, b):
    e = _merge22((a[0], a[2]), (b[0], b[2]))
    o = _merge22((a[1], a[3]), (b[1], b[3]))
    c1, c2 = _ce(o[0], e[1])
    c3, c4 = _ce(o[1], e[2])
    c5, c6 = _ce(o[2], e[3])
    return (e[0], c1, c2, c3, c4, c5, c6, o[3])


def _merge88_low(a, b):
    c = [jnp.minimum(a[i], b[7 - i]) for i in range(8)]
    for d in (4, 2, 1):
        nc = list(c)
        for i in range(8):
            if (i // d) % 2 == 0:
                nc[i], nc[i + d] = _ce(c[i], c[i + d])
        c = nc
    return tuple(c)


def _kth_smallest(work, l):
    r, kk = work.shape
    n_cols = kk // 128
    assert n_cols % 8 == 0 and l == 8
    cols = [work[:, i * 128:(i + 1) * 128] for i in range(n_cols)]
    s2 = [_ce(cols[2 * i], cols[2 * i + 1]) for i in range(n_cols // 2)]
    s4 = [_merge22(s2[2 * i], s2[2 * i + 1]) for i in range(len(s2) // 2)]
    s8 = [_merge44(s4[2 * i], s4[2 * i + 1]) for i in range(len(s4) // 2)]
    while len(s8) > 1:
        s8 = [_merge88_low(s8[2 * i], s8[2 * i + 1]) for i in range(len(s8) // 2)]
    cand = jnp.concatenate(s8[0], axis=1)
    pos_inf = jnp.float32(jnp.inf)
    for _ in range(l - 1):
        m = jnp.min(cand, axis=1, keepdims=True)
        cand = jnp.where(cand == m, pos_inf, cand)
    return jnp.min(cand, axis=1, keepdims=True)


def _fused_body(flat_ref, lin_wt_ref, dict_wt_ref, lin_b_ref,
                rep_ref, z_ref, loss_ref, perp_ref,
                counts_ref, sumsq_ref, *, n_total, n_blocks):
    i = pl.program_id(0)
    x = flat_ref[...]
    lin_wt = lin_wt_ref[...]
    dict_wt = dict_wt_ref[...]
    r, _ = x.shape
    k = lin_wt.shape[1]

    logits = jax.lax.dot_general(
        x, lin_wt, (((1,), (0,)), ((), ())),
        preferred_element_type=jnp.float32) + lin_b_ref[...]
    rowmax = jnp.max(logits, axis=1, keepdims=True)
    e = jnp.exp(logits - rowmax)
    denom = jnp.sum(e, axis=1, keepdims=True)

    dots = jax.lax.dot_general(
        x, dict_wt, (((1,), (0,)), ((), ())),
        preferred_element_type=jnp.float32)
    dsq = jnp.sum(dict_wt * dict_wt, axis=0, keepdims=True)
    fsq = jnp.sum(x * x, axis=1, keepdims=True)
    work = fsq + dsq - 2.0 * dots

    t8 = _kth_smallest(work, _L)
    maskf = jnp.where(work <= t8, jnp.float32(1.0), jnp.float32(0.0))
    colsum = jnp.sum(maskf, axis=0, keepdims=True)

    scale = 1.0 / (jnp.float32(_L) * denom)
    rep_ref[...] = maskf * (e * scale)

    @pl.when(i == 0)
    def _init():
        counts_ref[...] = jnp.zeros_like(counts_ref)
        sumsq_ref[0, 0] = jnp.float32(0.0)

    counts_ref[...] += colsum

    @pl.when(jnp.sum(colsum) > jnp.float32(r * _L) + 0.5)
    def _tie_fixup():
        iota = jax.lax.broadcasted_iota(jnp.int32, (r, k), 1)
        pos_inf = jnp.float32(jnp.inf)
        wk = work
        for _ in range(_L):
            m = jnp.min(wk, axis=1, keepdims=True)
            cand = jnp.where(wk == m, iota, k)
            first = jnp.min(cand, axis=1, keepdims=True)
            wk = jnp.where(iota == first, pos_inf, wk)
        acc = jnp.where(wk == pos_inf, jnp.float32(1.0), jnp.float32(0.0))
        rep_ref[...] = acc * (e * scale)
        counts_ref[...] += jnp.sum(acc, axis=0, keepdims=True) - colsum

    rep = rep_ref[...]
    z_dl = jax.lax.dot_general(
        rep, dict_wt, (((1,), (1,)), ((), ())),
        preferred_element_type=jnp.float32)
    diff = z_dl - x
    z_ref[...] = x + diff
    sumsq_ref[0, 0] += jnp.sum(diff * diff)

    @pl.when(i == n_blocks - 1)
    def _finalize():
        loss_ref[...] = jnp.full(
            (1, 1), (_BETA / (n_total * _DIM)) * sumsq_ref[0, 0], jnp.float32)
        avg = counts_ref[...] * (1.0 / n_total)
        p = avg / jnp.sum(avg)
        perp_ref[...] = jnp.full(
            (1, 1), jnp.exp(-jnp.sum(p * jnp.log(p + _EPS))), jnp.float32)


def kernel(z_e, dict_w, lin_w, lin_b):
    b, c, h, w = z_e.shape
    n = b * h * w
    k = dict_w.shape[0]
    flat = jnp.transpose(z_e, (0, 2, 3, 1)).reshape(n, c)
    lin_wt = lin_w.T
    dict_wt = dict_w.T
    lin_b2 = lin_b.reshape(1, k)

    r = 256 if n % 256 == 0 else n
    n_blocks = n // r

    rep_sparse, z_dl, loss, perp = pl.pallas_call(
        functools.partial(_fused_body, n_total=n, n_blocks=n_blocks),
        grid=(n_blocks,),
        in_specs=[
            pl.BlockSpec((r, c), lambda i: (i, 0)),
            pl.BlockSpec((c, k), lambda i: (0, 0)),
            pl.BlockSpec((c, k), lambda i: (0, 0)),
            pl.BlockSpec((1, k), lambda i: (0, 0)),
        ],
        out_specs=[
            pl.BlockSpec((r, k), lambda i: (i, 0)),
            pl.BlockSpec((r, c), lambda i: (i, 0)),
            pl.BlockSpec((1, 1), lambda i: (0, 0)),
            pl.BlockSpec((1, 1), lambda i: (0, 0)),
        ],
        out_shape=[
            jax.ShapeDtypeStruct((n, k), jnp.float32),
            jax.ShapeDtypeStruct((n, c), jnp.float32),
            jax.ShapeDtypeStruct((1, 1), jnp.float32),
            jax.ShapeDtypeStruct((1, 1), jnp.float32),
        ],
        scratch_shapes=[
            pltpu.VMEM((1, k), jnp.float32),
            pltpu.SMEM((1, 1), jnp.float32),
        ],
    )(flat, lin_wt, dict_wt, lin_b2)

    z_st = jnp.transpose(z_dl.reshape(b, h, w, c), (0, 3, 1, 2))
    return (loss.reshape(()), z_st, perp.reshape(()), rep_sparse)


# speedup vs baseline: 1.3049x; 1.1873x over previous
import functools

import jax
import jax.numpy as jnp
from jax.experimental import pallas as pl
from jax.experimental.pallas import tpu as pltpu

_DIM = 64
_L = 8
_BETA = 0.25
_EPS = 1e-10


def _ce(a, b):
    return jnp.minimum(a, b), jnp.maximum(a, b)


def _merge22(a, b):
    c0, t = _ce(a[0], b[0])
    s, c3 = _ce(a[1], b[1])
    c1, c2 = _ce(t, s)
    return (c0, c1, c2, c3)


def _merge44(a, b):
    e = _merge22((a[0], a[2]), (b[0], b[2]))
    o = _merge22((a[1], a[3]), (b[1], b[3]))
    c1, c2 = _ce(o[0], e[1])
    c3, c4 = _ce(o[1], e[2])
    c5, c6 = _ce(o[2], e[3])
    return (e[0], c1, c2, c3, c4, c5, c6, o[3])


def _merge88_low(a, b):
    c = [jnp.minimum(a[i], b[7 - i]) for i in range(8)]
    for d in (4, 2, 1):
        nc = list(c)
        for i in range(8):
            if (i // d) % 2 == 0:
                nc[i], nc[i + d] = _ce(c[i], c[i + d])
        c = nc
    return tuple(c)


def _kth_smallest(work, l):
    r, kk = work.shape
    n_cols = kk // 128
    assert n_cols % 8 == 0 and l == 8
    cols = [work[:, i * 128:(i + 1) * 128] for i in range(n_cols)]
    s2 = [_ce(cols[2 * i], cols[2 * i + 1]) for i in range(n_cols // 2)]
    s4 = [_merge22(s2[2 * i], s2[2 * i + 1]) for i in range(len(s2) // 2)]
    s8 = [_merge44(s4[2 * i], s4[2 * i + 1]) for i in range(len(s4) // 2)]
    while len(s8) > 1:
        s8 = [_merge88_low(s8[2 * i], s8[2 * i + 1]) for i in range(len(s8) // 2)]
    cand = jnp.concatenate(s8[0], axis=1)
    pos_inf = jnp.float32(jnp.inf)
    for _ in range(l - 1):
        m = jnp.min(cand, axis=1, keepdims=True)
        cand = jnp.where(cand == m, pos_inf, cand)
    return jnp.min(cand, axis=1, keepdims=True)


def _fused_body(flat_ref, lin_wt_ref, dict_wt_ref, lin_b_ref,
                rep_ref, z_ref, loss_ref, perp_ref,
                counts_ref, sumsq_ref, *, n_total, n_blocks):
    i = pl.program_id(0)
    x = flat_ref[...]
    lin_wt = lin_wt_ref[...]
    dict_wt = dict_wt_ref[...]
    r, _ = x.shape
    k = lin_wt.shape[1]

    logits = jax.lax.dot_general(
        x, lin_wt, (((1,), (0,)), ((), ())),
        preferred_element_type=jnp.float32) + lin_b_ref[...]
    rowmax = jnp.max(logits, axis=1, keepdims=True)
    e = jnp.exp(logits - rowmax)
    denom = jnp.sum(e, axis=1, keepdims=True)

    dots = jax.lax.dot_general(
        x, dict_wt, (((1,), (0,)), ((), ())),
        preferred_element_type=jnp.float32)
    dsq = jnp.sum(dict_wt * dict_wt, axis=0, keepdims=True)
    fsq = jnp.sum(x * x, axis=1, keepdims=True)
    work = fsq + dsq - 2.0 * dots

    t8 = _kth_smallest(work, _L)
    maskf = jnp.where(work <= t8, jnp.float32(1.0), jnp.float32(0.0))
    colsum = jnp.sum(maskf, axis=0, keepdims=True)

    scale = 1.0 / (jnp.float32(_L) * denom)
    rep = maskf * (e * scale)
    rep_ref[...] = rep

    z_dl = jax.lax.dot_general(
        rep, dict_wt, (((1,), (1,)), ((), ())),
        preferred_element_type=jnp.float32)
    diff = z_dl - x
    z_ref[...] = x + diff
    ssq = jnp.sum(diff * diff)

    @pl.when(i == 0)
    def _init():
        counts_ref[...] = jnp.zeros_like(counts_ref)
        sumsq_ref[0, 0] = jnp.float32(0.0)

    counts_ref[...] += colsum
    sumsq_ref[0, 0] += ssq

    @pl.when(jnp.sum(colsum) > jnp.float32(r * _L) + 0.5)
    def _tie_fixup():
        iota = jax.lax.broadcasted_iota(jnp.int32, (r, k), 1)
        pos_inf = jnp.float32(jnp.inf)
        wk = work
        for _ in range(_L):
            m = jnp.min(wk, axis=1, keepdims=True)
            cand = jnp.where(wk == m, iota, k)
            first = jnp.min(cand, axis=1, keepdims=True)
            wk = jnp.where(iota == first, pos_inf, wk)
        acc = jnp.where(wk == pos_inf, jnp.float32(1.0), jnp.float32(0.0))
        rep2 = acc * (e * scale)
        rep_ref[...] = rep2
        z2 = jax.lax.dot_general(
            rep2, dict_wt, (((1,), (1,)), ((), ())),
            preferred_element_type=jnp.float32)
        d2 = z2 - x
        z_ref[...] = x + d2
        counts_ref[...] += jnp.sum(acc, axis=0, keepdims=True) - colsum
        sumsq_ref[0, 0] += jnp.sum(d2 * d2) - ssq

    @pl.when(i == n_blocks - 1)
    def _finalize():
        loss_ref[...] = jnp.full(
            (1, 1), (_BETA / (n_total * _DIM)) * sumsq_ref[0, 0], jnp.float32)
        avg = counts_ref[...] * (1.0 / n_total)
        p = avg / jnp.sum(avg)
        perp_ref[...] = jnp.full(
            (1, 1), jnp.exp(-jnp.sum(p * jnp.log(p + _EPS))), jnp.float32)


def kernel(z_e, dict_w, lin_w, lin_b):
    b, c, h, w = z_e.shape
    n = b * h * w
    k = dict_w.shape[0]
    flat = jnp.transpose(z_e, (0, 2, 3, 1)).reshape(n, c)
    lin_wt = lin_w.T
    dict_wt = dict_w.T
    lin_b2 = lin_b.reshape(1, k)

    r = 256 if n % 256 == 0 else n
    n_blocks = n // r

    rep_sparse, z_dl, loss, perp = pl.pallas_call(
        functools.partial(_fused_body, n_total=n, n_blocks=n_blocks),
        grid=(n_blocks,),
        in_specs=[
            pl.BlockSpec((r, c), lambda i: (i, 0)),
            pl.BlockSpec((c, k), lambda i: (0, 0)),
            pl.BlockSpec((c, k), lambda i: (0, 0)),
            pl.BlockSpec((1, k), lambda i: (0, 0)),
        ],
        out_specs=[
            pl.BlockSpec((r, k), lambda i: (i, 0)),
            pl.BlockSpec((r, c), lambda i: (i, 0)),
            pl.BlockSpec((1, 1), lambda i: (0, 0)),
            pl.BlockSpec((1, 1), lambda i: (0, 0)),
        ],
        out_shape=[
            jax.ShapeDtypeStruct((n, k), jnp.float32),
            jax.ShapeDtypeStruct((n, c), jnp.float32),
            jax.ShapeDtypeStruct((1, 1), jnp.float32),
            jax.ShapeDtypeStruct((1, 1), jnp.float32),
        ],
        scratch_shapes=[
            pltpu.VMEM((1, k), jnp.float32),
            pltpu.SMEM((1, 1), jnp.float32),
        ],
    )(flat, lin_wt, dict_wt, lin_b2)

    z_st = jnp.transpose(z_dl.reshape(b, h, w, c), (0, 3, 1, 2))
    return (loss.reshape(()), z_st, perp.reshape(()), rep_sparse)
